# Initial kernel scaffold; baseline (speedup 1.0000x reference)
import jax
import jax.numpy as jnp
from jax.experimental import pallas as pl


def kernel(x, edge_index, edge_attr, W1, root1, bias1, W2, root2, bias2):
    raise NotImplementedError("write your pallas kernel here")


# speedup vs baseline: 3.0345x; 3.0345x over previous
import functools

import jax
import jax.numpy as jnp
from jax import lax
from jax.experimental import pallas as pl
from jax.experimental.pallas import tpu as pltpu
from jax.experimental.pallas import tpu_sc as plsc

N = 10000
E = 320000
IN_FEAT = 128
DIM = 16
EDIM = 4
NBASIS = 16
YW = NBASIS * DIM

NC = 2
NS = 16
NW = NC * NS
CH = 80
CHUNKS_PER_W = E // (NW * CH)
STRIPE = 1000
NSTRIPE = N // STRIPE


def _k1_body(x_ref, w_ref, r_ref, b_ref, y_ref, xr_ref):
    xv = x_ref[...]
    y_ref[...] = jnp.dot(xv, w_ref[...], preferred_element_type=jnp.float32)
    xr_ref[...] = (
        jnp.dot(xv, r_ref[...], preferred_element_type=jnp.float32) + b_ref[...]
    )


def _tc_expand1(x, w1f, root1, bias1):
    return pl.pallas_call(
        _k1_body,
        out_shape=(
            jax.ShapeDtypeStruct((N, YW), jnp.float32),
            jax.ShapeDtypeStruct((N, DIM), jnp.float32),
        ),
    )(x, w1f, root1, bias1)


def _k2_body(aggp, cntp, xr1, w_ref, r_ref, b_ref, y_ref, hr_ref, cnt_ref):
    agg = aggp[0] + aggp[1]
    cnt = jnp.maximum(cntp[0] + cntp[1], 1.0)
    h = agg / cnt + xr1[...]
    h = jnp.where(h > 0.0, h, jnp.exp(h) - 1.0)
    y_ref[...] = jnp.dot(h, w_ref[...], preferred_element_type=jnp.float32)
    hr_ref[...] = (
        jnp.dot(h, r_ref[...], preferred_element_type=jnp.float32) + b_ref[...]
    )
    cnt_ref[...] = cnt


def _tc_mid(aggp, cntp, xr1, w2f, root2, bias2):
    return pl.pallas_call(
        _k2_body,
        out_shape=(
            jax.ShapeDtypeStruct((N, YW), jnp.float32),
            jax.ShapeDtypeStruct((N, DIM), jnp.float32),
            jax.ShapeDtypeStruct((N, DIM), jnp.float32),
        ),
    )(aggp, cntp, xr1, w2f, root2, bias2)


def _k3_body(aggp, cnt, hr, out_ref):
    out_ref[...] = (aggp[0] + aggp[1]) / cnt[...] + hr[...]


def _tc_final(aggp, cnt, hr2):
    return pl.pallas_call(
        _k3_body,
        out_shape=jax.ShapeDtypeStruct((N, DIM), jnp.float32),
    )(aggp, cnt, hr2)


def _edge_butterfly(yrows, msg, i, u0, u1, u2, u3):
    y = [yrows[i, pl.ds(c * DIM, DIM)] for c in range(NBASIS)]
    t = [y[2 * j] + u0 * (y[2 * j + 1] - y[2 * j]) for j in range(8)]
    s = [t[2 * j] + u1 * (t[2 * j + 1] - t[2 * j]) for j in range(4)]
    r = [s[2 * j] + u2 * (s[2 * j + 1] - s[2 * j]) for j in range(2)]
    msg[i, :] = r[0] + u3 * (r[1] - r[0])


def _make_sc_pass(with_cnt):
    out_type = [jax.ShapeDtypeStruct((NC, N, DIM), jnp.float32)]
    scratch = [
        pltpu.VMEM((CH,), jnp.int32),
        pltpu.VMEM((CH,), jnp.int32),
        pltpu.VMEM((CH * EDIM,), jnp.float32),
        pltpu.VMEM((CH, YW), jnp.float32),
        pltpu.VMEM((CH, DIM), jnp.float32),
        pltpu.VMEM((STRIPE, DIM), jnp.float32),
        pltpu.SemaphoreType.DMA,
        pltpu.VMEM_SHARED((N, DIM), jnp.float32),
    ]
    if with_cnt:
        out_type.append(jax.ShapeDtypeStruct((NC, N, DIM), jnp.float32))
        scratch.append(pltpu.VMEM_SHARED((N, DIM), jnp.float32))
        scratch.append(pltpu.VMEM((CH, DIM), jnp.float32))

    def body(y_hbm, src_hbm, dst_hbm, ea_hbm, *rest):
        if with_cnt:
            (agg_out, cnt_out, sidx, didx, ea, yrows, msg, stage, sem,
             agg_sh, cnt_sh, ones) = rest
        else:
            (agg_out, sidx, didx, ea, yrows, msg, stage, sem, agg_sh) = rest
        cid = lax.axis_index("c")
        sid = lax.axis_index("s")
        wid = cid * NS + sid

        zv = jnp.zeros((DIM,), jnp.float32)

        def zrow(j, c):
            stage[j, :] = zv
            return c

        lax.fori_loop(0, STRIPE, zrow, 0)
        r0 = sid * STRIPE

        @pl.when(sid < NSTRIPE)
        def _init():
            pltpu.sync_copy(stage, agg_sh.at[pl.ds(r0, STRIPE)])
            if with_cnt:
                pltpu.sync_copy(stage, cnt_sh.at[pl.ds(r0, STRIPE)])

        if with_cnt:
            ov = jnp.ones((DIM,), jnp.float32)

            def orow(j, c):
                ones[j, :] = ov
                return c

            lax.fori_loop(0, CH, orow, 0)
        plsc.subcore_barrier()

        base0 = wid * (CH * CHUNKS_PER_W)

        def chunk_body(k, c):
            base = base0 + k * CH
            pltpu.sync_copy(src_hbm.at[pl.ds(base, CH)], sidx)
            pltpu.sync_copy(dst_hbm.at[pl.ds(base, CH)], didx)
            pltpu.sync_copy(ea_hbm.at[pl.ds(base * EDIM, CH * EDIM)], ea)
            pltpu.async_copy(y_hbm.at[sidx], yrows, sem).wait()

            def quad_body(q, cc):
                uv = ea[pl.ds(q * 16, 16)]
                uv = jnp.minimum(jnp.maximum(uv, 0.0), 1.0)
                for e in range(4):
                    i = q * 4 + e
                    _edge_butterfly(yrows, msg, i, uv[4 * e], uv[4 * e + 1],
                                    uv[4 * e + 2], uv[4 * e + 3])
                return cc

            lax.fori_loop(0, CH // 4, quad_body, 0)
            pltpu.sync_copy(msg, agg_sh.at[didx], add=True)
            if with_cnt:
                pltpu.sync_copy(ones, cnt_sh.at[didx], add=True)
            return c

        lax.fori_loop(0, CHUNKS_PER_W, chunk_body, 0)
        plsc.subcore_barrier()

        @pl.when(sid < NSTRIPE)
        def _copy_out():
            pltpu.sync_copy(
                agg_sh.at[pl.ds(r0, STRIPE)],
                agg_out.at[cid, pl.ds(r0, STRIPE)],
            )
            if with_cnt:
                pltpu.sync_copy(
                    cnt_sh.at[pl.ds(r0, STRIPE)],
                    cnt_out.at[cid, pl.ds(r0, STRIPE)],
                )

    mesh = plsc.VectorSubcoreMesh(core_axis_name="c", subcore_axis_name="s")
    return pl.kernel(
        body,
        out_type=tuple(out_type) if with_cnt else out_type[0],
        mesh=mesh,
        scratch_types=scratch,
        compiler_params=pltpu.CompilerParams(use_tc_tiling_on_sc=False),
    )


_sc_pass_cnt = _make_sc_pass(True)
_sc_pass = _make_sc_pass(False)


def kernel(x, edge_index, edge_attr, W1, root1, bias1, W2, root2, bias2):
    src = edge_index[0]
    dst = edge_index[1]
    ea_flat = edge_attr.reshape(E * EDIM)
    w1f = jnp.transpose(W1, (1, 0, 2)).reshape(IN_FEAT, YW)
    w2f = jnp.transpose(W2, (1, 0, 2)).reshape(DIM, YW)

    y1, xr1 = _tc_expand1(x, w1f, root1, bias1[None, :])
    agg1p, cntp = _sc_pass_cnt(y1, src, dst, ea_flat)
    y2, hr2, cnt = _tc_mid(agg1p, cntp, xr1, w2f, root2, bias2[None, :])
    agg2p = _sc_pass(y2, src, dst, ea_flat)
    return _tc_final(agg2p, cnt, hr2)


# speedup vs baseline: 4.4001x; 1.4500x over previous
import functools

import jax
import jax.numpy as jnp
from jax import lax
from jax.experimental import pallas as pl
from jax.experimental.pallas import tpu as pltpu
from jax.experimental.pallas import tpu_sc as plsc

N = 10000
E = 320000
IN_FEAT = 128
DIM = 16
EDIM = 4
NBASIS = 16
YW = NBASIS * DIM

NC = 2
NS = 16
NW = NC * NS
CH = 200
SLAB = 100
NSLAB = CH // SLAB
CHUNKS_PER_W = E // (NW * CH)
STRIPE = 1000
NSTRIPE = N // STRIPE

_YCOLS = [(2 * j + p) * DIM + d for j in range(8) for d in range(DIM)
          for p in (0, 1)]


def _k1_body(x_ref, w_ref, r_ref, b_ref, y_ref, xr_ref):
    xv = x_ref[...]
    y_ref[...] = jnp.dot(
        xv, w_ref[...], preferred_element_type=jnp.float32
    ).astype(jnp.bfloat16)
    xr_ref[...] = (
        jnp.dot(xv, r_ref[...], preferred_element_type=jnp.float32) + b_ref[...]
    )


def _tc_expand1(x, w1f, root1, bias1):
    return pl.pallas_call(
        _k1_body,
        out_shape=(
            jax.ShapeDtypeStruct((N, YW), jnp.bfloat16),
            jax.ShapeDtypeStruct((N, DIM), jnp.float32),
        ),
    )(x, w1f, root1, bias1)


def _k2_body(aggp, cntp, xr1, w_ref, r_ref, b_ref, y_ref, hr_ref, cnt_ref):
    agg = aggp[0] + aggp[1]
    cnt = jnp.maximum(cntp[0] + cntp[1], 1.0)
    h = agg / cnt + xr1[...]
    h = jnp.where(h > 0.0, h, jnp.exp(h) - 1.0)
    y_ref[...] = jnp.dot(
        h, w_ref[...], preferred_element_type=jnp.float32
    ).astype(jnp.bfloat16)
    hr_ref[...] = (
        jnp.dot(h, r_ref[...], preferred_element_type=jnp.float32) + b_ref[...]
    )
    cnt_ref[...] = cnt


def _tc_mid(aggp, cntp, xr1, w2f, root2, bias2):
    return pl.pallas_call(
        _k2_body,
        out_shape=(
            jax.ShapeDtypeStruct((N, YW), jnp.bfloat16),
            jax.ShapeDtypeStruct((N, DIM), jnp.float32),
            jax.ShapeDtypeStruct((N, DIM), jnp.float32),
        ),
    )(aggp, cntp, xr1, w2f, root2, bias2)


def _k3_body(aggp, cnt, hr, out_ref):
    out_ref[...] = (aggp[0] + aggp[1]) / cnt[...] + hr[...]


def _tc_final(aggp, cnt, hr2):
    return pl.pallas_call(
        _k3_body,
        out_shape=jax.ShapeDtypeStruct((N, DIM), jnp.float32),
    )(aggp, cnt, hr2)


def _edge_butterfly(yrows, msg, b, i, u0, u1, u2, u3):
    t = []
    for j in range(8):
        w = yrows[b, i, pl.ds(j * 16, 16)]
        a = lax.bitcast_convert_type(w << 16, jnp.float32)
        bb = lax.bitcast_convert_type(w & jnp.int32(-65536), jnp.float32)
        t.append(a + u0 * (bb - a))
    s = [t[2 * j] + u1 * (t[2 * j + 1] - t[2 * j]) for j in range(4)]
    r = [s[2 * j] + u2 * (s[2 * j + 1] - s[2 * j]) for j in range(2)]
    msg[b, i, :] = r[0] + u3 * (r[1] - r[0])


def _make_sc_pass(with_cnt):
    out_type = [jax.ShapeDtypeStruct((NC, N, DIM), jnp.float32)]
    scratch = [
        pltpu.VMEM((2, NSLAB, SLAB), jnp.int32),
        pltpu.VMEM((2, NSLAB, SLAB), jnp.int32),
        pltpu.VMEM((2, CH * EDIM), jnp.float32),
        pltpu.VMEM((2, CH, YW // 2), jnp.int32),
        pltpu.VMEM((2, CH, DIM), jnp.float32),
        pltpu.VMEM((STRIPE, DIM), jnp.float32),
        pltpu.SemaphoreType.DMA,
        pltpu.SemaphoreType.DMA,
        pltpu.VMEM_SHARED((N, DIM), jnp.float32),
    ]
    if with_cnt:
        out_type.append(jax.ShapeDtypeStruct((NC, N, DIM), jnp.float32))
        scratch.append(pltpu.VMEM_SHARED((N, DIM), jnp.float32))
        scratch.append(pltpu.VMEM((SLAB, DIM), jnp.float32))

    def body(y_hbm, src_hbm, dst_hbm, ea_hbm, *rest):
        if with_cnt:
            (agg_out, cnt_out, sidx, didx, ea, yrows, msg, stage, gsem, msem,
             agg_sh, cnt_sh, ones) = rest
        else:
            (agg_out, sidx, didx, ea, yrows, msg, stage, gsem, msem,
             agg_sh) = rest
        cid = lax.axis_index("c")
        sid = lax.axis_index("s")
        wid = cid * NS + sid

        zv = jnp.zeros((DIM,), jnp.float32)

        def zrow(j, c):
            stage[j, :] = zv
            return c

        lax.fori_loop(0, STRIPE, zrow, 0)
        r0 = sid * STRIPE

        @pl.when(sid < NSTRIPE)
        def _init():
            pltpu.sync_copy(stage, agg_sh.at[pl.ds(r0, STRIPE)])
            if with_cnt:
                pltpu.sync_copy(stage, cnt_sh.at[pl.ds(r0, STRIPE)])

        if with_cnt:
            ov = jnp.ones((DIM,), jnp.float32)

            def orow(j, c):
                ones[j, :] = ov
                return c

            lax.fori_loop(0, SLAB, orow, 0)
        plsc.subcore_barrier()

        base0 = wid * (CH * CHUNKS_PER_W)
        brow0 = base0 // SLAB
        NCH = CHUNKS_PER_W

        def issue_meta(j, b):
            base = base0 + j * CH
            brow = brow0 + j * NSLAB
            pltpu.async_copy(src_hbm.at[pl.ds(brow, NSLAB)], sidx.at[b], msem)
            pltpu.async_copy(dst_hbm.at[pl.ds(brow, NSLAB)], didx.at[b], msem)
            pltpu.async_copy(
                ea_hbm.at[pl.ds(base * EDIM, CH * EDIM)], ea.at[b], msem)

        def wait_meta(j, b):
            base = base0 + j * CH
            brow = brow0 + j * NSLAB
            pltpu.make_async_copy(
                src_hbm.at[pl.ds(brow, NSLAB)], sidx.at[b], msem).wait()
            pltpu.make_async_copy(
                dst_hbm.at[pl.ds(brow, NSLAB)], didx.at[b], msem).wait()
            pltpu.make_async_copy(
                ea_hbm.at[pl.ds(base * EDIM, CH * EDIM)], ea.at[b],
                msem).wait()

        def issue_gather(b):
            for j in range(NSLAB):
                pltpu.async_copy(
                    y_hbm.at[sidx.at[b, j]],
                    yrows.at[b, pl.ds(j * SLAB, SLAB)], gsem)

        def wait_gather(b):
            for j in range(NSLAB):
                pltpu.make_async_copy(
                    y_hbm.at[sidx.at[b, j]],
                    yrows.at[b, pl.ds(j * SLAB, SLAB)], gsem).wait()

        pltpu.sync_copy(src_hbm.at[pl.ds(brow0, NSLAB)], sidx.at[0])
        pltpu.sync_copy(dst_hbm.at[pl.ds(brow0, NSLAB)], didx.at[0])
        pltpu.sync_copy(ea_hbm.at[pl.ds(base0 * EDIM, CH * EDIM)], ea.at[0])
        issue_meta(1, 1)
        issue_gather(0)

        def chunk_body(k, c):
            b = k % 2
            b2 = (k + 1) % 2
            wait_gather(b)

            @pl.when(k + 1 < NCH)
            def _next_gather():
                wait_meta(k + 1, b2)
                issue_gather(b2)

            def quad_body(q, cc):
                uv = ea[b, pl.ds(q * 16, 16)]
                uv = jnp.minimum(jnp.maximum(uv, 0.0), 1.0)
                for e in range(4):
                    i = q * 4 + e
                    _edge_butterfly(yrows, msg, b, i, uv[4 * e],
                                    uv[4 * e + 1], uv[4 * e + 2],
                                    uv[4 * e + 3])
                return cc

            lax.fori_loop(0, CH // 4, quad_body, 0)
            for j in range(NSLAB):
                pltpu.sync_copy(msg.at[b, pl.ds(j * SLAB, SLAB)],
                                agg_sh.at[didx.at[b, j]], add=True)
                if with_cnt:
                    pltpu.sync_copy(ones, cnt_sh.at[didx.at[b, j]], add=True)

            @pl.when(k + 2 < NCH)
            def _next_meta():
                issue_meta(k + 2, b)

            return c

        lax.fori_loop(0, NCH, chunk_body, 0)
        plsc.subcore_barrier()

        @pl.when(sid < NSTRIPE)
        def _copy_out():
            pltpu.sync_copy(
                agg_sh.at[pl.ds(r0, STRIPE)],
                agg_out.at[cid, pl.ds(r0, STRIPE)],
            )
            if with_cnt:
                pltpu.sync_copy(
                    cnt_sh.at[pl.ds(r0, STRIPE)],
                    cnt_out.at[cid, pl.ds(r0, STRIPE)],
                )

    mesh = plsc.VectorSubcoreMesh(core_axis_name="c", subcore_axis_name="s")
    return pl.kernel(
        body,
        out_type=tuple(out_type) if with_cnt else out_type[0],
        mesh=mesh,
        scratch_types=scratch,
        compiler_params=pltpu.CompilerParams(use_tc_tiling_on_sc=False),
    )


_sc_pass_cnt = _make_sc_pass(True)
_sc_pass = _make_sc_pass(False)


def kernel(x, edge_index, edge_attr, W1, root1, bias1, W2, root2, bias2):
    src = edge_index[0].reshape(E // SLAB, SLAB)
    dst = edge_index[1].reshape(E // SLAB, SLAB)
    ea_flat = edge_attr.reshape(E * EDIM)
    ycols = jnp.asarray(_YCOLS, jnp.int32)
    w1f = jnp.transpose(W1, (1, 0, 2)).reshape(IN_FEAT, YW)[:, ycols]
    w2f = jnp.transpose(W2, (1, 0, 2)).reshape(DIM, YW)[:, ycols]

    y1, xr1 = _tc_expand1(x, w1f, root1, bias1[None, :])
    y1i = lax.bitcast_convert_type(y1.reshape(N, YW // 2, 2), jnp.int32)
    agg1p, cntp = _sc_pass_cnt(y1i, src, dst, ea_flat)
    y2, hr2, cnt = _tc_mid(agg1p, cntp, xr1, w2f, root2, bias2[None, :])
    y2i = lax.bitcast_convert_type(y2.reshape(N, YW // 2, 2), jnp.int32)
    agg2p = _sc_pass(y2i, src, dst, ea_flat)
    return _tc_final(agg2p, cnt, hr2)
